# baseline (speedup 1.0000x reference)
import functools

import jax
import jax.numpy as jnp
import numpy as np
from jax import lax
from jax.experimental import pallas as pl
from jax.experimental.pallas import tpu as pltpu
from jax.experimental.pallas import tpu_sc as plsc

N = 1 << 20
K = N >> 1
RADIX = 256
NSUB = 16
NCORE = 2
CHUNK = N // NSUB
SUB = CHUNK // 16
NWIN = 16
WINV = SUB // NWIN


def _np_threefry2x32(k0, k1, x0, x1):
    M = np.uint64(0xFFFFFFFF)
    x0 = np.uint64(x0) & M
    x1 = np.uint64(x1) & M
    ks = [np.uint64(k0), np.uint64(k1), np.uint64(k0 ^ k1 ^ 0x1BD11BDA)]
    rots = [[13, 15, 26, 6], [17, 29, 16, 24]]
    x0 = (x0 + ks[0]) & M
    x1 = (x1 + ks[1]) & M
    for i in range(5):
        for r in rots[i % 2]:
            x0 = (x0 + x1) & M
            x1 = ((x1 << np.uint64(r)) | (x1 >> np.uint64(32 - r))) & M
            x1 = x1 ^ x0
        x0 = (x0 + ks[(i + 1) % 3]) & M
        x1 = (x1 + ks[(i + 2) % 3] + np.uint64(i + 1)) & M
    return int(x0), int(x1)


def _round_subkeys():
    key = (0, 0)
    sub1 = _np_threefry2x32(key[0], key[1], 0, 1)
    new_key = _np_threefry2x32(key[0], key[1], 0, 0)
    sub2 = _np_threefry2x32(new_key[0], new_key[1], 0, 1)
    return sub1, sub2


_SUB1, _SUB2 = _round_subkeys()


_ROWS = N // 128
_BLK_ROWS = 1024


def _keygen_body(out_ref):
    s = pl.program_id(0)
    blk = pl.program_id(1)
    r = lax.broadcasted_iota(jnp.int32, (_BLK_ROWS, 128), 0)
    c = lax.broadcasted_iota(jnp.int32, (_BLK_ROWS, 128), 1)
    i = (blk * _BLK_ROWS + r) * 128 + c
    x1 = i.astype(jnp.uint32)
    x0 = jnp.zeros_like(x1)
    k0 = jnp.where(s == 0, np.uint32(_SUB1[0]), np.uint32(_SUB2[0]))
    k1 = jnp.where(s == 0, np.uint32(_SUB1[1]), np.uint32(_SUB2[1]))
    k2 = k0 ^ k1 ^ np.uint32(0x1BD11BDA)
    ks = [k0, k1, k2]
    rots = [[13, 15, 26, 6], [17, 29, 16, 24]]
    x0 = x0 + ks[0]
    x1 = x1 + ks[1]
    for it in range(5):
        for rr in rots[it % 2]:
            x0 = x0 + x1
            x1 = (x1 << np.uint32(rr)) | (x1 >> np.uint32(32 - rr))
            x1 = x1 ^ x0
        x0 = x0 + ks[(it + 1) % 3]
        x1 = x1 + ks[(it + 2) % 3] + np.uint32(it + 1)
    out_ref[...] = (x0 ^ x1).astype(jnp.int32)[None]


def _gen_keys():
    return pl.pallas_call(
        _keygen_body,
        out_shape=jax.ShapeDtypeStruct((2, _ROWS, 128), jnp.int32),
        grid=(2, _ROWS // _BLK_ROWS),
        out_specs=pl.BlockSpec((1, _BLK_ROWS, 128), lambda s, i: (s, i, 0)),
    )()


def _pass_body(shift, first, last, *refs):
    if first:
        (keys_in, keys_out, vals_out,
         keys_v, cur_v, scan_v, vidx_v, valt_v, kstg_v, dstg_v, vstg_v,
         gt_sh) = refs
        vals_in = None
    elif last:
        (keys_in, vals_in, vals_out,
         keys_v, cur_v, scan_v, vidx_v, valt_v, kstg_v, dstg_v, vstg_v,
         gt_sh) = refs
        keys_out = None
    else:
        (keys_in, vals_in, keys_out, vals_out,
         keys_v, cur_v, scan_v, vidx_v, valt_v, kstg_v, dstg_v, vstg_v,
         gt_sh) = refs
    c = lax.axis_index("c")
    w = lax.axis_index("s")
    lane = lax.iota(jnp.int32, 16)
    ones = jnp.ones((16,), jnp.int32)
    sbase = c * N
    cbase = sbase + w * CHUNK

    pltpu.sync_copy(keys_in.at[pl.ds(cbase, CHUNK)], keys_v)
    z = jnp.zeros((16,), jnp.int32)

    def zbody(t, _):
        cur_v[pl.ds(t * 16, 16)] = z
        return 0
    lax.fori_loop(0, RADIX, zbody, 0)

    def digits_of(kv):
        return (lax.shift_right_logical(kv, jnp.int32(shift))
                & jnp.int32(RADIX - 1))

    def hbody(t, _):
        kv = plsc.load_gather(keys_v, [lane * SUB + t])
        plsc.addupdate_scatter(cur_v, [digits_of(kv) * 16 + lane], ones)
        return 0
    lax.fori_loop(0, SUB, hbody, 0)

    pltpu.sync_copy(cur_v, gt_sh.at[w])
    plsc.subcore_barrier()
    wsel = (lane == w).astype(jnp.int32)

    def dig_body(d, p_run):
        blk = d // 32
        dd = d % 32

        @pl.when(dd == 0)
        def _():
            pltpu.sync_copy(gt_sh.at[:, pl.ds(blk * 512, 512)], scan_v)
        acc = jnp.zeros((16,), jnp.int32)
        for l in range(16):
            acc = acc + plsc.load_gather(
                scan_v, [lane, jnp.broadcast_to(dd * 16 + l, (16,))])
        total = lax.reduce_sum_p.bind(acc, axes=(0,))
        csum_w = plsc.cumsum(acc)
        ecw = lax.reduce_sum_p.bind((csum_w - acc) * wsel, axes=(0,))
        row = plsc.load_gather(
            scan_v, [jnp.broadcast_to(w, (16,)), dd * 16 + lane])
        ecl = plsc.cumsum(row) - row
        cur_v[pl.ds(d * 16, 16)] = (p_run + ecw) + ecl
        return p_run + total
    lax.fori_loop(0, RADIX, dig_body, jnp.int32(0))

    def win_body(win, _):
        v0 = win * WINV

        def vb_body(t, _):
            vidx_v[pl.ds(t * 16, 16)] = cbase + lane * SUB + (v0 + t)
            return 0
        lax.fori_loop(0, WINV, vb_body, 0)
        if not first:
            pltpu.sync_copy(vals_in.at[vidx_v], valt_v)

        def t_body(t, _):
            kv = plsc.load_gather(keys_v, [lane * SUB + (v0 + t)])
            cidx = digits_of(kv) * 16 + lane
            dest = plsc.load_gather(cur_v, [cidx])
            plsc.store_scatter(cur_v, [cidx], dest + 1)
            if first:
                val = cbase - sbase + lane * SUB + (v0 + t)
            else:
                val = valt_v[pl.ds(t * 16, 16)]
            if not last:
                kstg_v[pl.ds(t * 16, 16)] = kv
            dstg_v[pl.ds(t * 16, 16)] = sbase + dest
            vstg_v[pl.ds(t * 16, 16)] = val
            return 0
        lax.fori_loop(0, WINV, t_body, 0)
        if not last:
            pltpu.sync_copy(kstg_v, keys_out.at[dstg_v])
        pltpu.sync_copy(vstg_v, vals_out.at[dstg_v])
        return 0
    lax.fori_loop(0, NWIN, win_body, 0)


def _sort_scratch():
    return [
        pltpu.VMEM((CHUNK,), jnp.int32),
        pltpu.VMEM((RADIX * 16,), jnp.int32),
        pltpu.VMEM((16, 512), jnp.int32),
        pltpu.VMEM((WINV * 16,), jnp.int32),
        pltpu.VMEM((WINV * 16,), jnp.int32),
        pltpu.VMEM((WINV * 16,), jnp.int32),
        pltpu.VMEM((WINV * 16,), jnp.int32),
        pltpu.VMEM((WINV * 16,), jnp.int32),
        pltpu.VMEM_SHARED((16, RADIX * 16), jnp.int32),
    ]


def _make_pass(shift, first, last, n_out):
    mesh = plsc.VectorSubcoreMesh(core_axis_name="c", subcore_axis_name="s")
    return pl.kernel(
        functools.partial(_pass_body, shift, first, last),
        out_type=[jax.ShapeDtypeStruct((2 * N,), jnp.int32)
                  for _ in range(n_out)],
        mesh=mesh,
        compiler_params=pltpu.CompilerParams(
            needs_layout_passes=False, use_tc_tiling_on_sc=False),
        scratch_types=_sort_scratch(),
    )


def _dual_argsort(keys_flat):
    ka, va = _make_pass(0, True, False, 2)(keys_flat)
    kb, vb = _make_pass(8, False, False, 2)(ka, va)
    ka2, va2 = _make_pass(16, False, False, 2)(kb, vb)
    (sigma,) = _make_pass(24, False, True, 1)(ka2, va2)
    return sigma


_CQ = K // (NCORE * NSUB)
_CW = 2048


def _compose_body(sigma, out, idx_v, gat_v):
    c = lax.axis_index("c")
    w = lax.axis_index("s")
    wid = w * NCORE + c
    base = wid * _CQ
    for win in range(_CQ // _CW):
        qb = base + win * _CW
        pltpu.sync_copy(sigma.at[pl.ds(N + qb, _CW)], idx_v)
        pltpu.sync_copy(sigma.at[idx_v], gat_v)
        pltpu.sync_copy(gat_v, out.at[pl.ds(qb, _CW)])


def _compose(sigma):
    mesh = plsc.VectorSubcoreMesh(core_axis_name="c", subcore_axis_name="s")
    f = pl.kernel(
        _compose_body,
        out_type=jax.ShapeDtypeStruct((K,), jnp.int32),
        mesh=mesh,
        compiler_params=pltpu.CompilerParams(
            needs_layout_passes=False, use_tc_tiling_on_sc=False),
        scratch_types=[
            pltpu.VMEM((_CW,), jnp.int32),
            pltpu.VMEM((_CW,), jnp.int32),
        ],
    )
    return f(sigma)


def kernel(pos, batch):
    del batch
    assert pos.shape[0] == N
    keys = _gen_keys().reshape(2 * N)
    sigma = _dual_argsort(keys)
    return _compose(sigma)


# speedup vs baseline: 1.0216x; 1.0216x over previous
import functools

import jax
import jax.numpy as jnp
import numpy as np
from jax import lax
from jax.experimental import pallas as pl
from jax.experimental.pallas import tpu as pltpu
from jax.experimental.pallas import tpu_sc as plsc

N = 1 << 20
K = N >> 1
RADIX = 256
NSUB = 16
NCORE = 2
CHUNK = N // NSUB
SUB = CHUNK // 16
NWIN = 16
WINV = SUB // NWIN
UNROLL = 4


def _np_threefry2x32(k0, k1, x0, x1):
    M = np.uint64(0xFFFFFFFF)
    x0 = np.uint64(x0) & M
    x1 = np.uint64(x1) & M
    ks = [np.uint64(k0), np.uint64(k1), np.uint64(k0 ^ k1 ^ 0x1BD11BDA)]
    rots = [[13, 15, 26, 6], [17, 29, 16, 24]]
    x0 = (x0 + ks[0]) & M
    x1 = (x1 + ks[1]) & M
    for i in range(5):
        for r in rots[i % 2]:
            x0 = (x0 + x1) & M
            x1 = ((x1 << np.uint64(r)) | (x1 >> np.uint64(32 - r))) & M
            x1 = x1 ^ x0
        x0 = (x0 + ks[(i + 1) % 3]) & M
        x1 = (x1 + ks[(i + 2) % 3] + np.uint64(i + 1)) & M
    return int(x0), int(x1)


def _round_subkeys():
    key = (0, 0)
    sub1 = _np_threefry2x32(key[0], key[1], 0, 1)
    new_key = _np_threefry2x32(key[0], key[1], 0, 0)
    sub2 = _np_threefry2x32(new_key[0], new_key[1], 0, 1)
    return sub1, sub2


_SUB1, _SUB2 = _round_subkeys()


_ROWS = N // 128
_BLK_ROWS = 1024


def _keygen_body(out_ref):
    s = pl.program_id(0)
    blk = pl.program_id(1)
    r = lax.broadcasted_iota(jnp.int32, (_BLK_ROWS, 128), 0)
    c = lax.broadcasted_iota(jnp.int32, (_BLK_ROWS, 128), 1)
    p = (blk * _BLK_ROWS + r) * 128 + c
    tp = p & (CHUNK - 1)
    i = (p - tp) + (tp & 15) * SUB + (tp >> 4)
    x1 = i.astype(jnp.uint32)
    x0 = jnp.zeros_like(x1)
    k0 = jnp.where(s == 0, np.uint32(_SUB1[0]), np.uint32(_SUB2[0]))
    k1 = jnp.where(s == 0, np.uint32(_SUB1[1]), np.uint32(_SUB2[1]))
    k2 = k0 ^ k1 ^ np.uint32(0x1BD11BDA)
    ks = [k0, k1, k2]
    rots = [[13, 15, 26, 6], [17, 29, 16, 24]]
    x0 = x0 + ks[0]
    x1 = x1 + ks[1]
    for it in range(5):
        for rr in rots[it % 2]:
            x0 = x0 + x1
            x1 = (x1 << np.uint32(rr)) | (x1 >> np.uint32(32 - rr))
            x1 = x1 ^ x0
        x0 = x0 + ks[(it + 1) % 3]
        x1 = x1 + ks[(it + 2) % 3] + np.uint32(it + 1)
    out_ref[...] = (x0 ^ x1).astype(jnp.int32)[None]


def _gen_keys():
    return pl.pallas_call(
        _keygen_body,
        out_shape=jax.ShapeDtypeStruct((2, _ROWS, 128), jnp.int32),
        grid=(2, _ROWS // _BLK_ROWS),
        out_specs=pl.BlockSpec((1, _BLK_ROWS, 128), lambda s, i: (s, i, 0)),
    )()


def _pass_body(shift, first, last, *refs):
    if first:
        (keys_in, keys_out, vals_out,
         keys_v, cur_v, scan_v, valt_v, kstg_v, dstg_v, vstg_v, sems,
         gt_sh) = refs
        vals_in = None
    elif last:
        (keys_in, vals_in, vals_out,
         keys_v, cur_v, scan_v, valt_v, kstg_v, dstg_v, vstg_v, sems,
         gt_sh) = refs
        keys_out = None
    else:
        (keys_in, vals_in, keys_out, vals_out,
         keys_v, cur_v, scan_v, valt_v, kstg_v, dstg_v, vstg_v, sems,
         gt_sh) = refs
    c = lax.axis_index("c")
    w = lax.axis_index("s")
    lane = lax.iota(jnp.int32, 16)
    ones = jnp.ones((16,), jnp.int32)
    sbase = c * N
    cbase = sbase + w * CHUNK

    pltpu.sync_copy(keys_in.at[pl.ds(cbase, CHUNK)], keys_v)
    z = jnp.zeros((16,), jnp.int32)

    def zbody(t, _):
        cur_v[pl.ds(t * 16, 16)] = z
        return 0
    lax.fori_loop(0, RADIX, zbody, 0)

    def digits_of(kv):
        return (lax.shift_right_logical(kv, jnp.int32(shift))
                & jnp.int32(RADIX - 1))

    def hbody(t, _):
        for u in range(UNROLL):
            kv = keys_v[pl.ds((t * UNROLL + u) * 16, 16)]
            plsc.addupdate_scatter(cur_v, [digits_of(kv) * 16 + lane], ones)
        return 0
    lax.fori_loop(0, SUB // UNROLL, hbody, 0)

    pltpu.sync_copy(cur_v, gt_sh.at[w])
    plsc.subcore_barrier()
    wsel = (lane == w).astype(jnp.int32)

    def dig_body(d, p_run):
        blk = d // 32
        dd = d % 32

        @pl.when(dd == 0)
        def _():
            pltpu.sync_copy(gt_sh.at[:, pl.ds(blk * 512, 512)], scan_v)
        acc = jnp.zeros((16,), jnp.int32)
        for l in range(16):
            acc = acc + plsc.load_gather(
                scan_v, [lane, jnp.broadcast_to(dd * 16 + l, (16,))])
        total = lax.reduce_sum_p.bind(acc, axes=(0,))
        csum_w = plsc.cumsum(acc)
        ecw = lax.reduce_sum_p.bind((csum_w - acc) * wsel, axes=(0,))
        row = plsc.load_gather(
            scan_v, [jnp.broadcast_to(w, (16,)), dd * 16 + lane])
        ecl = plsc.cumsum(row) - row
        cur_v[pl.ds(d * 16, 16)] = (p_run + ecw) + ecl
        return p_run + total
    lax.fori_loop(0, RADIX, dig_body, jnp.int32(0))

    def transposed(dest):
        dc = dest & (CHUNK - 1)
        return sbase + (dest - dc) + (dc & (SUB - 1)) * 16 + (dc >> 12)

    val_hands = [None, None]
    scat_hands = [[], []]
    if not first:
        pltpu.sync_copy(vals_in.at[pl.ds(cbase, WINV * 16)], valt_v[0])
    for win in range(NWIN):
        par = win & 1
        npar = 1 - par
        if (not first) and win + 1 < NWIN:
            val_hands[npar] = pltpu.async_copy(
                vals_in.at[pl.ds(cbase + (win + 1) * WINV * 16, WINV * 16)],
                valt_v[npar], sems[2 + npar])
        v0 = win * WINV
        for h in scat_hands[par]:
            h.wait()
        scat_hands[par] = []
        kstg, dstg, vstg, vt = kstg_v[par], dstg_v[par], vstg_v[par], valt_v[par]

        def t_body(t, _):
            for u in range(UNROLL):
                tt = t * UNROLL + u
                kv = keys_v[pl.ds((v0 + tt) * 16, 16)]
                cidx = digits_of(kv) * 16 + lane
                dest = plsc.load_gather(cur_v, [cidx])
                plsc.store_scatter(cur_v, [cidx], dest + 1)
                if first:
                    val = (cbase - sbase) + lane * SUB + (v0 + tt)
                else:
                    val = vt[pl.ds(tt * 16, 16)]
                if not last:
                    kstg[pl.ds(tt * 16, 16)] = kv
                    dstg[pl.ds(tt * 16, 16)] = transposed(dest)
                else:
                    dstg[pl.ds(tt * 16, 16)] = sbase + dest
                vstg[pl.ds(tt * 16, 16)] = val
            return 0
        lax.fori_loop(0, WINV // UNROLL, t_body, 0)
        if not last:
            scat_hands[par].append(pltpu.async_copy(
                kstg, keys_out.at[dstg], sems[0]))
        scat_hands[par].append(pltpu.async_copy(
            vstg, vals_out.at[dstg], sems[1]))
        if (not first) and win + 1 < NWIN:
            val_hands[npar].wait()
    for par in range(2):
        for h in scat_hands[par]:
            h.wait()


def _sort_scratch():
    return [
        pltpu.VMEM((CHUNK,), jnp.int32),
        pltpu.VMEM((RADIX * 16,), jnp.int32),
        pltpu.VMEM((16, 512), jnp.int32),
        [pltpu.VMEM((WINV * 16,), jnp.int32)] * 2,
        [pltpu.VMEM((WINV * 16,), jnp.int32)] * 2,
        [pltpu.VMEM((WINV * 16,), jnp.int32)] * 2,
        [pltpu.VMEM((WINV * 16,), jnp.int32)] * 2,
        [pltpu.SemaphoreType.DMA] * 4,
        pltpu.VMEM_SHARED((16, RADIX * 16), jnp.int32),
    ]


def _make_pass(shift, first, last, n_out):
    mesh = plsc.VectorSubcoreMesh(core_axis_name="c", subcore_axis_name="s")
    return pl.kernel(
        functools.partial(_pass_body, shift, first, last),
        out_type=[jax.ShapeDtypeStruct((2 * N,), jnp.int32)
                  for _ in range(n_out)],
        mesh=mesh,
        compiler_params=pltpu.CompilerParams(
            needs_layout_passes=False, use_tc_tiling_on_sc=False),
        scratch_types=_sort_scratch(),
    )


def _dual_argsort(keys_flat):
    ka, va = _make_pass(0, True, False, 2)(keys_flat)
    kb, vb = _make_pass(8, False, False, 2)(ka, va)
    ka2, va2 = _make_pass(16, False, False, 2)(kb, vb)
    (sigma,) = _make_pass(24, False, True, 1)(ka2, va2)
    return sigma


_CQ = K // (NCORE * NSUB)
_CW = 2048


def _compose_body(sigma, out, idx_v, gat_v):
    c = lax.axis_index("c")
    w = lax.axis_index("s")
    wid = w * NCORE + c
    base = wid * _CQ
    for win in range(_CQ // _CW):
        qb = base + win * _CW
        pltpu.sync_copy(sigma.at[pl.ds(N + qb, _CW)], idx_v)
        pltpu.sync_copy(sigma.at[idx_v], gat_v)
        pltpu.sync_copy(gat_v, out.at[pl.ds(qb, _CW)])


def _compose(sigma):
    mesh = plsc.VectorSubcoreMesh(core_axis_name="c", subcore_axis_name="s")
    f = pl.kernel(
        _compose_body,
        out_type=jax.ShapeDtypeStruct((K,), jnp.int32),
        mesh=mesh,
        compiler_params=pltpu.CompilerParams(
            needs_layout_passes=False, use_tc_tiling_on_sc=False),
        scratch_types=[
            pltpu.VMEM((_CW,), jnp.int32),
            pltpu.VMEM((_CW,), jnp.int32),
        ],
    )
    return f(sigma)


def kernel(pos, batch):
    del batch
    assert pos.shape[0] == N
    keys = _gen_keys().reshape(2 * N)
    sigma = _dual_argsort(keys)
    return _compose(sigma)
